# Initial kernel scaffold; baseline (speedup 1.0000x reference)
import jax
import jax.numpy as jnp
from jax.experimental import pallas as pl


def kernel(virtual_values):
    raise NotImplementedError("write your pallas kernel here")


# speedup vs baseline: 35.2694x; 35.2694x over previous
import functools

import jax
import jax.numpy as jnp
from jax import lax
from jax.experimental import pallas as pl
from jax.experimental.pallas import tpu as pltpu

B = 4096
N = 20000

RB1 = 256
CB1 = 2048
NR1 = B // RB1
NC1 = (N + CB1 - 1) // CB1

RB2 = 256
CB2 = 2048
NR2 = B // RB2
NC2 = (N + CB2 - 1) // CB2

NEG_INF = float("-inf")
BIG_I32 = 2**31 - 1


def _top2_body(x_ref, idx_ref, pay_ref, m_s, s_s, i_s):
    c = pl.program_id(1)

    @pl.when(c == 0)
    def _init():
        m_s[...] = jnp.full(m_s.shape, NEG_INF, m_s.dtype)
        s_s[...] = jnp.full(s_s.shape, NEG_INF, s_s.dtype)
        i_s[...] = jnp.zeros(i_s.shape, i_s.dtype)

    x = x_ref[...]
    gcol = c * CB1 + lax.broadcasted_iota(jnp.int32, (RB1, CB1), 1)
    valid = gcol < N
    x = jnp.where(valid, x, NEG_INF)

    m_blk = jnp.max(x, axis=1, keepdims=True)
    idx_blk = jnp.min(jnp.where(x == m_blk, gcol, BIG_I32), axis=1, keepdims=True)
    s_blk = jnp.max(jnp.where(gcol == idx_blk, NEG_INF, x), axis=1, keepdims=True)

    m_run, s_run, i_run = m_s[...], s_s[...], i_s[...]
    take_new = m_blk > m_run
    m_s[...] = jnp.maximum(m_run, m_blk)
    s_s[...] = jnp.maximum(jnp.maximum(s_run, s_blk), jnp.minimum(m_run, m_blk))
    i_s[...] = jnp.where(take_new, idx_blk, i_run)

    @pl.when(c == NC1 - 1)
    def _emit():
        idx_ref[...] = jnp.broadcast_to(i_s[...], idx_ref.shape)
        pay_ref[...] = jnp.broadcast_to(jnp.maximum(s_s[...], 0.0), pay_ref.shape)


def _fill_body(idx_ref, pay_ref, alloc_ref, pay_out_ref):
    c = pl.program_id(1)
    winner = idx_ref[:, :1]
    payment = pay_ref[:, :1]
    gcol = c * CB2 + lax.broadcasted_iota(jnp.int32, (RB2, CB2), 1)
    onehot = gcol == winner
    alloc_ref[...] = jnp.where(onehot, jnp.float32(1.0), jnp.float32(0.0))
    pay_out_ref[...] = jnp.where(onehot, payment, jnp.float32(0.0))


@jax.jit
def kernel(virtual_values):
    idx, pay = pl.pallas_call(
        _top2_body,
        grid=(NR1, NC1),
        in_specs=[pl.BlockSpec((RB1, CB1), lambda r, c: (r, c))],
        out_specs=[
            pl.BlockSpec((RB1, 128), lambda r, c: (r, 0)),
            pl.BlockSpec((RB1, 128), lambda r, c: (r, 0)),
        ],
        out_shape=[
            jax.ShapeDtypeStruct((B, 128), jnp.int32),
            jax.ShapeDtypeStruct((B, 128), jnp.float32),
        ],
        scratch_shapes=[
            pltpu.VMEM((RB1, 1), jnp.float32),
            pltpu.VMEM((RB1, 1), jnp.float32),
            pltpu.VMEM((RB1, 1), jnp.int32),
        ],
        compiler_params=pltpu.CompilerParams(
            dimension_semantics=("parallel", "arbitrary"),
        ),
    )(virtual_values)

    allocations, payments = pl.pallas_call(
        _fill_body,
        grid=(NR2, NC2),
        in_specs=[
            pl.BlockSpec((RB2, 128), lambda r, c: (r, 0)),
            pl.BlockSpec((RB2, 128), lambda r, c: (r, 0)),
        ],
        out_specs=[
            pl.BlockSpec((RB2, CB2), lambda r, c: (r, c)),
            pl.BlockSpec((RB2, CB2), lambda r, c: (r, c)),
        ],
        out_shape=[
            jax.ShapeDtypeStruct((B, N), jnp.float32),
            jax.ShapeDtypeStruct((B, N), jnp.float32),
        ],
        compiler_params=pltpu.CompilerParams(
            dimension_semantics=("parallel", "parallel"),
        ),
    )(idx, pay)

    return (allocations, payments)


# speedup vs baseline: 36.9214x; 1.0468x over previous
import jax
import jax.numpy as jnp
from jax import lax
from jax.experimental import pallas as pl
from jax.experimental.pallas import tpu as pltpu

B = 4096
N = 20000

RB = 256
CB = 2048
NR = B // RB
NC = (N + CB - 1) // CB

NEG_INF = float("-inf")
BIG_I32 = 2**31 - 1


def _fused_body(x_ref, alloc_ref, pay_ref, m_s, s_s, i_s, fi_s, fp_s):
    r = pl.program_id(0)
    c = pl.program_id(1)
    gcol = c * CB + lax.broadcasted_iota(jnp.int32, (RB, CB), 1)

    @pl.when(r >= 1)
    def _fill():
        onehot = gcol == fi_s[...]
        alloc_ref[...] = jnp.where(onehot, jnp.float32(1.0), jnp.float32(0.0))
        pay_ref[...] = jnp.where(onehot, fp_s[...], jnp.float32(0.0))

    @pl.when(r < NR)
    def _reduce():
        @pl.when(c == 0)
        def _init():
            m_s[...] = jnp.full(m_s.shape, NEG_INF, m_s.dtype)
            s_s[...] = jnp.full(s_s.shape, NEG_INF, s_s.dtype)
            i_s[...] = jnp.zeros(i_s.shape, i_s.dtype)

        x = jnp.where(gcol < N, x_ref[...], NEG_INF)
        m_blk = jnp.max(x, axis=1, keepdims=True)
        idx_blk = jnp.min(jnp.where(x == m_blk, gcol, BIG_I32), axis=1,
                          keepdims=True)
        s_blk = jnp.max(jnp.where(gcol == idx_blk, NEG_INF, x), axis=1,
                        keepdims=True)

        m_run, s_run, i_run = m_s[...], s_s[...], i_s[...]
        m_s[...] = jnp.maximum(m_run, m_blk)
        s_s[...] = jnp.maximum(jnp.maximum(s_run, s_blk),
                               jnp.minimum(m_run, m_blk))
        i_s[...] = jnp.where(m_blk > m_run, idx_blk, i_run)

        @pl.when(c == NC - 1)
        def _finalize():
            fi_s[...] = i_s[...]
            fp_s[...] = jnp.maximum(s_s[...], 0.0)


@jax.jit
def kernel(virtual_values):
    allocations, payments = pl.pallas_call(
        _fused_body,
        grid=(NR + 1, NC),
        in_specs=[
            pl.BlockSpec(
                (RB, CB),
                lambda r, c: (jnp.minimum(r, NR - 1),
                              jnp.where(r < NR, c, NC - 1)),
            ),
        ],
        out_specs=[
            pl.BlockSpec(
                (RB, CB),
                lambda r, c: (jnp.maximum(r - 1, 0),
                              jnp.where(r >= 1, c, 0)),
            ),
            pl.BlockSpec(
                (RB, CB),
                lambda r, c: (jnp.maximum(r - 1, 0),
                              jnp.where(r >= 1, c, 0)),
            ),
        ],
        out_shape=[
            jax.ShapeDtypeStruct((B, N), jnp.float32),
            jax.ShapeDtypeStruct((B, N), jnp.float32),
        ],
        scratch_shapes=[
            pltpu.VMEM((RB, 1), jnp.float32),
            pltpu.VMEM((RB, 1), jnp.float32),
            pltpu.VMEM((RB, 1), jnp.int32),
            pltpu.VMEM((RB, 1), jnp.int32),
            pltpu.VMEM((RB, 1), jnp.float32),
        ],
        compiler_params=pltpu.CompilerParams(
            dimension_semantics=("arbitrary", "arbitrary"),
        ),
    )(virtual_values)

    return (allocations, payments)


# speedup vs baseline: 37.8717x; 1.0257x over previous
import jax
import jax.numpy as jnp
from jax import lax
from jax.experimental import pallas as pl
from jax.experimental.pallas import tpu as pltpu

B = 4096
N = 20000

RB = 512
CB = 2048
NR = B // RB
NC = (N + CB - 1) // CB

NEG_INF = float("-inf")
BIG_I32 = 2**31 - 1


def _fused_body(x_ref, alloc_ref, pay_ref, m_s, s_s, i_s, fi_s, fp_s):
    r = pl.program_id(0)
    c = pl.program_id(1)
    gcol = c * CB + lax.broadcasted_iota(jnp.int32, (RB, CB), 1)

    @pl.when(r >= 1)
    def _fill():
        onehot = gcol == fi_s[...]
        alloc_ref[...] = jnp.where(onehot, jnp.float32(1.0), jnp.float32(0.0))
        pay_ref[...] = jnp.where(onehot, fp_s[...], jnp.float32(0.0))

    @pl.when(r < NR)
    def _reduce():
        @pl.when(c == 0)
        def _init():
            m_s[...] = jnp.full(m_s.shape, NEG_INF, m_s.dtype)
            s_s[...] = jnp.full(s_s.shape, NEG_INF, s_s.dtype)
            i_s[...] = jnp.zeros(i_s.shape, i_s.dtype)

        x = jnp.where(gcol < N, x_ref[...], NEG_INF)
        m_blk = jnp.max(x, axis=1, keepdims=True)
        idx_blk = jnp.min(jnp.where(x == m_blk, gcol, BIG_I32), axis=1,
                          keepdims=True)
        s_blk = jnp.max(jnp.where(gcol == idx_blk, NEG_INF, x), axis=1,
                        keepdims=True)

        m_run, s_run, i_run = m_s[...], s_s[...], i_s[...]
        m_s[...] = jnp.maximum(m_run, m_blk)
        s_s[...] = jnp.maximum(jnp.maximum(s_run, s_blk),
                               jnp.minimum(m_run, m_blk))
        i_s[...] = jnp.where(m_blk > m_run, idx_blk, i_run)

        @pl.when(c == NC - 1)
        def _finalize():
            fi_s[...] = i_s[...]
            fp_s[...] = jnp.maximum(s_s[...], 0.0)


@jax.jit
def kernel(virtual_values):
    allocations, payments = pl.pallas_call(
        _fused_body,
        grid=(NR + 1, NC),
        in_specs=[
            pl.BlockSpec(
                (RB, CB),
                lambda r, c: (jnp.minimum(r, NR - 1),
                              jnp.where(r < NR, c, NC - 1)),
            ),
        ],
        out_specs=[
            pl.BlockSpec(
                (RB, CB),
                lambda r, c: (jnp.maximum(r - 1, 0),
                              jnp.where(r >= 1, c, 0)),
            ),
            pl.BlockSpec(
                (RB, CB),
                lambda r, c: (jnp.maximum(r - 1, 0),
                              jnp.where(r >= 1, c, 0)),
            ),
        ],
        out_shape=[
            jax.ShapeDtypeStruct((B, N), jnp.float32),
            jax.ShapeDtypeStruct((B, N), jnp.float32),
        ],
        scratch_shapes=[
            pltpu.VMEM((RB, 1), jnp.float32),
            pltpu.VMEM((RB, 1), jnp.float32),
            pltpu.VMEM((RB, 1), jnp.int32),
            pltpu.VMEM((RB, 1), jnp.int32),
            pltpu.VMEM((RB, 1), jnp.float32),
        ],
        compiler_params=pltpu.CompilerParams(
            dimension_semantics=("arbitrary", "arbitrary"),
        ),
    )(virtual_values)

    return (allocations, payments)
